# baseline (device time: 103300 ns/iter reference)
import jax
import jax.numpy as jnp
from jax import lax
from jax.experimental import pallas as pl
from jax.experimental.pallas import tpu as pltpu

N_DEV = 4


def kernel(x, w_mat, scale_x, scale_w):
    m_per, k = x.shape
    _, n_per = w_mat.shape
    half = m_per // 2

    wq = w_mat.astype(jnp.bfloat16)

    def body(x_ref, w_ref, sx_ref, sw_ref, out_ref,
             buf_a, buf_b, stage, outblk,
             stage_sem, out_sems, send_a, recv_a, send_b, recv_b):
        my = lax.axis_index("i")
        left = lax.rem(my + N_DEV - 1, N_DEV)
        right = lax.rem(my + 1, N_DEV)

        barrier = pltpu.get_barrier_semaphore()
        for nbr in (left, right):
            pl.semaphore_signal(
                barrier, inc=1,
                device_id=(nbr,), device_id_type=pl.DeviceIdType.MESH,
            )
        pl.semaphore_wait(barrier, 2)

        cp_a = pltpu.make_async_copy(
            x_ref.at[pl.ds(0, half), :], stage, stage_sem)
        cp_a.start()
        cp_a.wait()
        buf_a[0] = stage[...].astype(jnp.float8_e4m3fn)

        def make_rdma(h):
            rdma_a = pltpu.make_async_remote_copy(
                src_ref=buf_a.at[h],
                dst_ref=buf_a.at[h + 1],
                send_sem=send_a.at[h],
                recv_sem=recv_a.at[h],
                device_id=(right,),
                device_id_type=pl.DeviceIdType.MESH,
            )
            rdma_b = pltpu.make_async_remote_copy(
                src_ref=buf_b.at[h],
                dst_ref=buf_b.at[h + 1],
                send_sem=send_b.at[h],
                recv_sem=recv_b.at[h],
                device_id=(left,),
                device_id_type=pl.DeviceIdType.MESH,
            )
            return rdma_a, rdma_b

        rdma_a0, rdma_b0 = make_rdma(0)
        rdma_a0.start()

        cp_b = pltpu.make_async_copy(
            x_ref.at[pl.ds(half, half), :], stage, stage_sem)
        cp_b.start()
        cp_b.wait()
        buf_b[0] = stage[...].astype(jnp.float8_e4m3fn)
        rdma_b0.start()

        scale = sx_ref[0, 0] * sw_ref[0, 0]
        out_copies = []

        def compute_slot(s):
            for origin, buf in (
                (lax.rem(my - s + N_DEV, N_DEV), buf_a),
                (lax.rem(my + s, N_DEV), buf_b),
            ):
                blk = len(out_copies)
                slot = blk % 2
                if blk >= 2:
                    out_copies[blk - 2].wait()
                acc = jnp.dot(buf[s].astype(jnp.bfloat16), w_ref[...],
                              preferred_element_type=jnp.float32)
                outblk[slot] = jnp.maximum(acc * scale, 0.0)
                row0 = origin * m_per + (0 if buf is buf_a else half)
                cp = pltpu.make_async_copy(
                    outblk.at[slot],
                    out_ref.at[pl.ds(row0, half), :],
                    out_sems.at[slot],
                )
                cp.start()
                out_copies.append(cp)

        for h in range(N_DEV - 1):
            if h > 0:
                rdma_a, rdma_b = make_rdma(h)
                rdma_a.start()
                rdma_b.start()
            else:
                rdma_a, rdma_b = rdma_a0, rdma_b0
            compute_slot(h)
            rdma_a.wait()
            rdma_b.wait()
        compute_slot(N_DEV - 1)

        out_copies[-2].wait()
        out_copies[-1].wait()

    return pl.pallas_call(
        body,
        out_shape=jax.ShapeDtypeStruct((N_DEV * m_per, n_per), jnp.float32),
        in_specs=[
            pl.BlockSpec(memory_space=pl.ANY),
            pl.BlockSpec(memory_space=pltpu.VMEM),
            pl.BlockSpec(memory_space=pltpu.SMEM),
            pl.BlockSpec(memory_space=pltpu.SMEM),
        ],
        out_specs=pl.BlockSpec(memory_space=pl.ANY),
        scratch_shapes=[
            pltpu.VMEM((N_DEV, half, k), jnp.float8_e4m3fn),
            pltpu.VMEM((N_DEV, half, k), jnp.float8_e4m3fn),
            pltpu.VMEM((half, k), jnp.float32),
            pltpu.VMEM((2, half, n_per), jnp.float32),
            pltpu.SemaphoreType.DMA,
            pltpu.SemaphoreType.DMA((2,)),
            pltpu.SemaphoreType.DMA((N_DEV - 1,)),
            pltpu.SemaphoreType.DMA((N_DEV - 1,)),
            pltpu.SemaphoreType.DMA((N_DEV - 1,)),
            pltpu.SemaphoreType.DMA((N_DEV - 1,)),
        ],
        compiler_params=pltpu.CompilerParams(
            collective_id=0,
            vmem_limit_bytes=100 * 1024 * 1024,
        ),
    )(x, wq, scale_x.reshape(1, 1), scale_w.reshape(1, 1))


# device time: 44807 ns/iter; 2.3054x vs baseline; 2.3054x over previous
import jax
import jax.numpy as jnp
from jax import lax
from jax.experimental import pallas as pl
from jax.experimental.pallas import tpu as pltpu

N_DEV = 4


def kernel(x, w_mat, scale_x, scale_w):
    m_per, k = x.shape
    _, n_per = w_mat.shape
    half = m_per // 2

    wq = w_mat.astype(jnp.bfloat16)

    def body(x_ref, w_ref, sx_ref, sw_ref, out_ref,
             buf_a, buf_b, stage, outblk,
             stage_sem, out_sems, send_a, recv_a, send_b, recv_b):
        my = lax.axis_index("i")
        left = lax.rem(my + N_DEV - 1, N_DEV)
        right = lax.rem(my + 1, N_DEV)

        with jax.named_scope("barrier"):
            barrier = pltpu.get_barrier_semaphore()
            for nbr in (left, right):
                pl.semaphore_signal(
                    barrier, inc=1,
                    device_id=(nbr,), device_id_type=pl.DeviceIdType.MESH,
                )
            pl.semaphore_wait(barrier, 2)

        with jax.named_scope("stageA"):
            cp_a = pltpu.make_async_copy(
                x_ref.at[pl.ds(0, half), :], stage, stage_sem)
            cp_a.start()
            cp_a.wait()
            buf_a[0] = stage[...].astype(jnp.float8_e4m3fn)

        def make_rdma(h):
            rdma_a = pltpu.make_async_remote_copy(
                src_ref=buf_a.at[h],
                dst_ref=buf_a.at[h + 1],
                send_sem=send_a.at[h],
                recv_sem=recv_a.at[h],
                device_id=(right,),
                device_id_type=pl.DeviceIdType.MESH,
            )
            rdma_b = pltpu.make_async_remote_copy(
                src_ref=buf_b.at[h],
                dst_ref=buf_b.at[h + 1],
                send_sem=send_b.at[h],
                recv_sem=recv_b.at[h],
                device_id=(left,),
                device_id_type=pl.DeviceIdType.MESH,
            )
            return rdma_a, rdma_b

        rdma_a0, rdma_b0 = make_rdma(0)
        rdma_a0.start()

        with jax.named_scope("stageB"):
            cp_b = pltpu.make_async_copy(
                x_ref.at[pl.ds(half, half), :], stage, stage_sem)
            cp_b.start()
            cp_b.wait()
            buf_b[0] = stage[...].astype(jnp.float8_e4m3fn)
            rdma_b0.start()

        scale = sx_ref[0, 0] * sw_ref[0, 0]
        out_copies = []

        def compute_slot(s):
            for origin, buf in (
                (lax.rem(my - s + N_DEV, N_DEV), buf_a),
                (lax.rem(my + s, N_DEV), buf_b),
            ):
                blk = len(out_copies)
                slot = blk % 2
                if blk >= 2:
                    out_copies[blk - 2].wait()
                acc = jnp.dot(buf[s].astype(jnp.bfloat16), w_ref[...],
                              preferred_element_type=jnp.float32)
                outblk[slot] = jnp.maximum(acc * scale, 0.0)
                row0 = origin * m_per + (0 if buf is buf_a else half)
                cp = pltpu.make_async_copy(
                    outblk.at[slot],
                    out_ref.at[pl.ds(row0, half), :],
                    out_sems.at[slot],
                )
                cp.start()
                out_copies.append(cp)

        for h in range(N_DEV - 1):
            if h > 0:
                with jax.named_scope(f"send#hop={h}"):
                    rdma_a, rdma_b = make_rdma(h)
                    rdma_a.start()
                    rdma_b.start()
            else:
                rdma_a, rdma_b = rdma_a0, rdma_b0
            with jax.named_scope(f"compute#hop={h}"):
                compute_slot(h)
            with jax.named_scope(f"wait#hop={h}"):
                rdma_a.wait()
                rdma_b.wait()
        with jax.named_scope("tail_compute"):
            compute_slot(N_DEV - 1)

        with jax.named_scope("drain"):
            out_copies[-2].wait()
            out_copies[-1].wait()

    return pl.pallas_call(
        body,
        out_shape=jax.ShapeDtypeStruct((N_DEV * m_per, n_per), jnp.float32),
        in_specs=[
            pl.BlockSpec(memory_space=pl.ANY),
            pl.BlockSpec(memory_space=pltpu.VMEM),
            pl.BlockSpec(memory_space=pltpu.SMEM),
            pl.BlockSpec(memory_space=pltpu.SMEM),
        ],
        out_specs=pl.BlockSpec(memory_space=pl.ANY),
        scratch_shapes=[
            pltpu.VMEM((N_DEV, half, k), jnp.float8_e4m3fn),
            pltpu.VMEM((N_DEV, half, k), jnp.float8_e4m3fn),
            pltpu.VMEM((half, k), jnp.float32),
            pltpu.VMEM((2, half, n_per), jnp.float32),
            pltpu.SemaphoreType.DMA,
            pltpu.SemaphoreType.DMA((2,)),
            pltpu.SemaphoreType.DMA((N_DEV - 1,)),
            pltpu.SemaphoreType.DMA((N_DEV - 1,)),
            pltpu.SemaphoreType.DMA((N_DEV - 1,)),
            pltpu.SemaphoreType.DMA((N_DEV - 1,)),
        ],
        compiler_params=pltpu.CompilerParams(
            collective_id=0,
            vmem_limit_bytes=100 * 1024 * 1024,
        ),
    )(x, wq, scale_x.reshape(1, 1), scale_w.reshape(1, 1))
